# baseline (speedup 1.0000x reference)
import functools

import jax
import jax.numpy as jnp
from jax import lax
from jax.experimental import pallas as pl
from jax.experimental.pallas import tpu as pltpu
from jax.experimental.pallas import tpu_sc as plsc

_NL = 1000000
_D = 64
_B = 16384
_N = 2 * _B

_NW = 32
_SEL_W = 31360
_CAP = 2048
_W = 128
_NCHUNK = 248
_TAIL_LO = 999936
_LAST_L0 = _TAIL_LO - _W

_LN2 = 0.6931471805599453
_C3 = 0.3333333432674408
_C5 = 0.2
_C7 = 0.14285714285714285


def _softplus16(x):
    t = jnp.exp(jnp.minimum(x, 20.0))
    z = 1.0 + t
    zi = lax.bitcast_convert_type(z, jnp.int32)
    e = lax.shift_right_arithmetic(zi - 0x3F3504F3, 23)
    m = lax.bitcast_convert_type(zi - lax.shift_left(e, 23), jnp.float32)
    s = (m - 1.0) / (m + 1.0)
    s2 = s * s
    p = 2.0 * s * (1.0 + s2 * (_C3 + s2 * (_C5 + s2 * _C7)))
    ln_z = e.astype(jnp.float32) * _LN2 + p
    return jnp.where(x > 20.0, x, ln_z)


def _iota16():
    return jnp.arange(16, dtype=jnp.int32)


def _make_scan_extract():
    mesh = plsc.VectorSubcoreMesh(core_axis_name="c", subcore_axis_name="s")

    @functools.partial(
        pl.kernel,
        mesh=mesh,
        compiler_params=pltpu.CompilerParams(
            use_tc_tiling_on_sc=True, needs_layout_passes=False),
        out_type=jax.ShapeDtypeStruct((_N, 2 * _D), jnp.float32),
        scratch_types=[
            pltpu.VMEM((1024,), jnp.int32),
            pltpu.VMEM((_CAP + 16,), jnp.int32),
            pltpu.VMEM((_CAP + 16,), jnp.int32),
            pltpu.VMEM((_CAP + 16,), jnp.int32),
            pltpu.VMEM((_CAP + 16,), jnp.int32),
            pltpu.VMEM((256,), jnp.int32),
            pltpu.VMEM((3, _D, _W), jnp.float32),
            pltpu.VMEM((3, _D, _W), jnp.float32),
            pltpu.VMEM((2, 64, 2 * _D), jnp.float32),
            pltpu.VMEM((2, 64), jnp.int32),
            pltpu.VMEM((_D, _NL - _TAIL_LO), jnp.float32),
            pltpu.VMEM((_D, _NL - _TAIL_LO), jnp.float32),
            pltpu.SemaphoreType.DMA,
            pltpu.SemaphoreType.DMA,
            pltpu.SemaphoreType.DMA,
        ],
    )
    def scan_extract(idxf, u_t, v_t, u_tail, v_tail, i_out, ibuf, ilist,
                     nlist, clist, cnlist, hist, uslab, vslab,
                     obuf, nbuf, utailbuf, vtailbuf, semu, semv, semf):
        wid = lax.axis_index("s") * 2 + lax.axis_index("c")
        sel_lo = wid * _SEL_W
        sel_hi = jnp.minimum(sel_lo + _SEL_W, _NL)
        iota = _iota16()
        pltpu.sync_copy(u_tail, utailbuf)
        pltpu.sync_copy(v_tail, vtailbuf)

        def scan_select(woff):
            def piece(p, carry):
                off, cbase = carry
                pltpu.sync_copy(idxf.at[pl.ds(p * 1024, 1024)], ibuf)

                def vec(k, carry2):
                    off2, cb2 = carry2
                    v = ibuf[pl.ds(16 * k, 16)]
                    m = (v >= sel_lo) & (v < sel_hi)
                    mi = m.astype(jnp.int32)
                    cnt = plsc.all_reduce_population_count(m)[0]
                    rank = cb2 + plsc.cumsum(mi) - 1
                    m2 = m & (rank >= woff) & (rank < woff + _CAP)
                    nvec = p * 1024 + 16 * k + iota
                    plsc.store_compressed(ilist.at[pl.ds(off2, 16)], v, mask=m2)
                    plsc.store_compressed(nlist.at[pl.ds(off2, 16)], nvec, mask=m2)
                    adv = plsc.all_reduce_population_count(m2)[0]
                    return off2 + adv, cb2 + cnt

                return lax.fori_loop(0, 64, vec, (off, cbase), unroll=2)

            off, total = lax.fori_loop(0, 32, piece, (jnp.int32(0), jnp.int32(0)))
            del off
            return total

        def sort_entries(n_entries):
            zero = jnp.zeros((16,), jnp.int32)
            for k in range(16):
                hist[pl.ds(16 * k, 16)] = zero
            nvecs = (n_entries + 15) // 16

            def hvec(k, carry):
                iv = ilist[pl.ds(16 * k, 16)]
                valid = (16 * k + iota) < n_entries
                cid = jnp.where(iv >= _TAIL_LO, _NCHUNK,
                                lax.shift_right_logical(iv - sel_lo, 7))
                plsc.addupdate_scatter(hist, [cid], jnp.ones((16,), jnp.int32),
                                       mask=valid)
                return carry

            lax.fori_loop(0, nvecs, hvec, jnp.int32(0))

            def pvec(k, carry):
                h = hist[pl.ds(16 * k, 16)]
                c = plsc.cumsum(h)
                excl = carry + c - h
                hist[pl.ds(16 * k, 16)] = excl
                return carry + c[15]

            lax.fori_loop(0, 16, pvec, jnp.int32(0))

            def place(e, carry):
                i = ilist[pl.ds(e, 16)][0]
                n = nlist[pl.ds(e, 16)][0]
                cid = jnp.where(i >= _TAIL_LO, _NCHUNK,
                                lax.shift_right_logical(i - sel_lo, 7))
                pos = hist[pl.ds(cid, 16)][0]
                lane0 = iota == 0
                plsc.store_scatter(clist, [jnp.full((16,), pos, jnp.int32)],
                                   jnp.full((16,), i, jnp.int32), mask=lane0)
                plsc.store_scatter(cnlist, [jnp.full((16,), pos, jnp.int32)],
                                   jnp.full((16,), n, jnp.int32), mask=lane0)
                plsc.store_scatter(hist, [jnp.full((16,), cid, jnp.int32)],
                                   jnp.full((16,), pos + 1, jnp.int32),
                                   mask=lane0)
                return carry

            lax.fori_loop(0, n_entries, place, jnp.int32(0))

        def extract_entries(cid, l0, slot, usrc, vsrc):
            lo = jnp.where(cid > 0, hist[pl.ds(jnp.maximum(cid - 1, 0), 16)][0],
                           0)
            cnt_hi = hist[pl.ds(cid, 16)][0]
            cnt = cnt_hi - lo

            def flush(st):
                sl, f = st
                bank = f & 1
                pltpu.make_async_copy(
                    obuf.at[bank], i_out.at[nbuf.at[bank]], semf).start()

                @pl.when(f >= 1)
                def _():
                    pltpu.make_async_copy(
                        obuf.at[1 - bank], i_out.at[nbuf.at[1 - bank]],
                        semf).wait()

                return jnp.int32(0), f + 1

            lane0 = iota == 0

            def ent(e, st):
                sl, f = st
                bank = f & 1
                i = clist[pl.ds(e, 16)][0]
                n = cnlist[pl.ds(e, 16)][0]
                lv = jnp.full((16,), i - l0, jnp.int32)
                for k in range(4):
                    cvec = 16 * k + iota
                    u16 = plsc.load_gather(usrc, [cvec, lv])
                    v16 = plsc.load_gather(vsrc, [cvec, lv])
                    obuf[bank, sl, pl.ds(16 * k, 16)] = u16
                    obuf[bank, sl, pl.ds(_D + 16 * k, 16)] = _softplus16(v16)
                plsc.store_scatter(nbuf.at[bank],
                                   [jnp.full((16,), sl, jnp.int32)],
                                   jnp.full((16,), n, jnp.int32), mask=lane0)
                sl = sl + 1
                return lax.cond(sl == 64, flush, lambda s: s, (sl, f))

            return lax.fori_loop(lo, cnt_hi, ent, slot)

        def process_round(woff, total):
            n_entries = jnp.minimum(total - woff, _CAP)
            sort_entries(n_entries)
            slot = (jnp.int32(0), jnp.int32(0))

            def slab_copies(c, bank):
                raw = sel_lo + c * _W
                l0 = pl.multiple_of(jnp.minimum(raw, _LAST_L0), 128)
                csl = pl.ds(l0, _W)
                return [
                    pltpu.make_async_copy(u_t.at[:, csl], uslab.at[bank], semu),
                    pltpu.make_async_copy(v_t.at[:, csl], vslab.at[bank], semv),
                ]

            for h in slab_copies(jnp.int32(0), jnp.int32(0)):
                h.start()
            for h in slab_copies(jnp.int32(1), jnp.int32(1)):
                h.start()

            def chunk(c, sl):
                bank = jnp.remainder(c, 3)
                for h in slab_copies(c, bank):
                    h.wait()
                raw = sel_lo + c * _W
                l0 = pl.multiple_of(jnp.minimum(raw, _LAST_L0), 128)
                sl = extract_entries(c, l0, sl, uslab.at[bank],
                                     vslab.at[bank])

                @pl.when(c + 2 < _NCHUNK)
                def _():
                    for h in slab_copies(c + 2, jnp.remainder(c + 2, 3)):
                        h.start()

                return sl

            slot = lax.fori_loop(0, _NCHUNK, chunk, slot)

            slot = extract_entries(jnp.int32(_NCHUNK), jnp.int32(_TAIL_LO),
                                   slot, utailbuf, vtailbuf)

            sl_end, f_end = slot

            @pl.when(f_end >= 1)
            def _():
                bank = (f_end - 1) & 1
                pltpu.make_async_copy(
                    obuf.at[bank], i_out.at[nbuf.at[bank]], semf).wait()

            @pl.when(sl_end > 0)
            def _():
                lane0 = iota == 0
                bank = f_end & 1
                n0 = nbuf[bank, pl.ds(0, 16)][0]

                def pad(p, carry):
                    plsc.store_scatter(
                        nbuf.at[bank], [jnp.full((16,), p, jnp.int32)],
                        jnp.full((16,), n0, jnp.int32), mask=lane0)
                    for k in range(8):
                        obuf[bank, p, pl.ds(16 * k, 16)] = (
                            obuf[bank, 0, pl.ds(16 * k, 16)])
                    return carry

                lax.fori_loop(sl_end, 64, pad, jnp.int32(0))
                pltpu.sync_copy(obuf.at[bank], i_out.at[nbuf.at[bank]])

        total = scan_select(jnp.int32(0))
        process_round(jnp.int32(0), total)

        def extra_round(r, tot):
            @pl.when(r * _CAP < tot)
            def _():
                t2 = scan_select(r * _CAP)
                process_round(r * _CAP, t2)

            return tot

        lax.fori_loop(1, _N // _CAP, extra_round, total)

    return scan_extract


def _tc_transpose(inter):
    i3 = inter.reshape(2, _B, 2 * _D)

    def body(i_ref, z_ref):
        blk = i_ref[0]
        for s in range(2):
            z_ref[0, s] = blk[:, s * _D:(s + 1) * _D].T

    return pl.pallas_call(
        body,
        grid=(2, _B // 512),
        in_specs=[pl.BlockSpec((1, 512, 2 * _D), lambda i2, b: (i2, b, 0))],
        out_specs=pl.BlockSpec((1, 2, _D, 512), lambda i2, b: (i2, 0, 0, b)),
        out_shape=jax.ShapeDtypeStruct((2, 2, _D, _B), jnp.float32),
    )(i3)


def kernel(idxs, U, V):
    idxf = jnp.transpose(idxs.astype(jnp.int32)).reshape(_N)
    u_t = jnp.transpose(U)
    v_t = jnp.transpose(V)
    u_tail = u_t[:, _TAIL_LO:]
    v_tail = v_t[:, _TAIL_LO:]
    inter = _make_scan_extract()(idxf, u_t, v_t, u_tail, v_tail)
    z = _tc_transpose(inter)
    return jnp.transpose(z, (3, 0, 1, 2))


# speedup vs baseline: 1.0983x; 1.0983x over previous
import functools

import jax
import jax.numpy as jnp
from jax import lax
from jax.experimental import pallas as pl
from jax.experimental.pallas import tpu as pltpu
from jax.experimental.pallas import tpu_sc as plsc

_NL = 1000000
_D = 64
_B = 16384
_N = 2 * _B

_NW = 32
_SEL_W = 31360
_CAP = 2048
_W = 256
_NCHUNK = 124
_TAIL_LO = 999936
_LAST_L0 = _TAIL_LO - _W

_LN2 = 0.6931471805599453
_C3 = 0.3333333432674408
_C5 = 0.2
_C7 = 0.14285714285714285


def _softplus16(x):
    t = jnp.exp(jnp.minimum(x, 20.0))
    z = 1.0 + t
    zi = lax.bitcast_convert_type(z, jnp.int32)
    e = lax.shift_right_arithmetic(zi - 0x3F3504F3, 23)
    m = lax.bitcast_convert_type(zi - lax.shift_left(e, 23), jnp.float32)
    s = (m - 1.0) / (m + 1.0)
    s2 = s * s
    p = 2.0 * s * (1.0 + s2 * (_C3 + s2 * (_C5 + s2 * _C7)))
    ln_z = e.astype(jnp.float32) * _LN2 + p
    return jnp.where(x > 20.0, x, ln_z)


def _iota16():
    return jnp.arange(16, dtype=jnp.int32)


def _make_scan_extract():
    mesh = plsc.VectorSubcoreMesh(core_axis_name="c", subcore_axis_name="s")

    @functools.partial(
        pl.kernel,
        mesh=mesh,
        compiler_params=pltpu.CompilerParams(
            use_tc_tiling_on_sc=True, needs_layout_passes=False),
        out_type=jax.ShapeDtypeStruct((_N, 2 * _D), jnp.float32),
        scratch_types=[
            pltpu.VMEM((4096,), jnp.int32),
            pltpu.VMEM((_CAP + 16,), jnp.int32),
            pltpu.VMEM((_CAP + 16,), jnp.int32),
            pltpu.VMEM((_CAP + 16,), jnp.int32),
            pltpu.VMEM((_CAP + 16,), jnp.int32),
            pltpu.VMEM((256,), jnp.int32),
            pltpu.VMEM((_D, _W), jnp.float32),
            pltpu.VMEM((_D, _W), jnp.float32),
            pltpu.VMEM((_D, _W), jnp.float32),
            pltpu.VMEM((_D, _W), jnp.float32),
            pltpu.VMEM((2, 128, 2 * _D), jnp.float32),
            pltpu.VMEM((2, 128), jnp.int32),
            pltpu.VMEM((_D, _NL - _TAIL_LO), jnp.float32),
            pltpu.VMEM((_D, _NL - _TAIL_LO), jnp.float32),
            pltpu.SemaphoreType.DMA,
            pltpu.SemaphoreType.DMA,
            pltpu.SemaphoreType.DMA,
        ],
    )
    def scan_extract(idxf, u_t, v_t, u_tail, v_tail, i_out, ibuf, ilist,
                     nlist, clist, cnlist, hist, ubuf0, vbuf0, ubuf1, vbuf1,
                     obuf, nbuf, utailbuf, vtailbuf, semu, semv, semf):
        wid = lax.axis_index("s") * 2 + lax.axis_index("c")
        sel_lo = wid * _SEL_W
        sel_hi = jnp.minimum(sel_lo + _SEL_W, _NL)
        iota = _iota16()
        pltpu.sync_copy(u_tail, utailbuf)
        pltpu.sync_copy(v_tail, vtailbuf)

        def scan_select(woff):
            def piece(p, carry):
                off, cbase = carry
                pltpu.sync_copy(idxf.at[pl.ds(p * 4096, 4096)], ibuf)

                def vec(k, carry2):
                    off2, cb2 = carry2
                    v = ibuf[pl.ds(16 * k, 16)]
                    m = (v >= sel_lo) & (v < sel_hi)
                    mi = m.astype(jnp.int32)
                    cnt = plsc.all_reduce_population_count(m)[0]
                    rank = cb2 + plsc.cumsum(mi) - 1
                    m2 = m & (rank >= woff) & (rank < woff + _CAP)
                    nvec = p * 4096 + 16 * k + iota
                    plsc.store_compressed(ilist.at[pl.ds(off2, 16)], v, mask=m2)
                    plsc.store_compressed(nlist.at[pl.ds(off2, 16)], nvec, mask=m2)
                    adv = plsc.all_reduce_population_count(m2)[0]
                    return off2 + adv, cb2 + cnt

                return lax.fori_loop(0, 256, vec, (off, cbase), unroll=2)

            off, total = lax.fori_loop(0, 8, piece, (jnp.int32(0), jnp.int32(0)))
            del off
            return total

        def sort_entries(n_entries):
            zero = jnp.zeros((16,), jnp.int32)
            for k in range(16):
                hist[pl.ds(16 * k, 16)] = zero
            nvecs = (n_entries + 15) // 16

            def hvec(k, carry):
                iv = ilist[pl.ds(16 * k, 16)]
                valid = (16 * k + iota) < n_entries
                cid = jnp.where(iv >= _TAIL_LO, _NCHUNK,
                                lax.shift_right_logical(iv - sel_lo, 8))
                plsc.addupdate_scatter(hist, [cid], jnp.ones((16,), jnp.int32),
                                       mask=valid)
                return carry

            lax.fori_loop(0, nvecs, hvec, jnp.int32(0))

            def pvec(k, carry):
                h = hist[pl.ds(16 * k, 16)]
                c = plsc.cumsum(h)
                excl = carry + c - h
                hist[pl.ds(16 * k, 16)] = excl
                return carry + c[15]

            lax.fori_loop(0, 16, pvec, jnp.int32(0))

            def place(e, carry):
                i = ilist[pl.ds(e, 16)][0]
                n = nlist[pl.ds(e, 16)][0]
                cid = jnp.where(i >= _TAIL_LO, _NCHUNK,
                                lax.shift_right_logical(i - sel_lo, 8))
                pos = hist[pl.ds(cid, 16)][0]
                lane0 = iota == 0
                plsc.store_scatter(clist, [jnp.full((16,), pos, jnp.int32)],
                                   jnp.full((16,), i, jnp.int32), mask=lane0)
                plsc.store_scatter(cnlist, [jnp.full((16,), pos, jnp.int32)],
                                   jnp.full((16,), n, jnp.int32), mask=lane0)
                plsc.store_scatter(hist, [jnp.full((16,), cid, jnp.int32)],
                                   jnp.full((16,), pos + 1, jnp.int32),
                                   mask=lane0)
                return carry

            lax.fori_loop(0, n_entries, place, jnp.int32(0))

        def extract_entries(cid, l0, slot, usrc, vsrc):
            lo = jnp.where(cid > 0, hist[pl.ds(jnp.maximum(cid - 1, 0), 16)][0],
                           0)
            cnt_hi = hist[pl.ds(cid, 16)][0]
            cnt = cnt_hi - lo

            def flush(st):
                sl, f = st
                bank = f & 1
                pltpu.make_async_copy(
                    obuf.at[bank], i_out.at[nbuf.at[bank]], semf).start()

                @pl.when(f >= 1)
                def _():
                    pltpu.make_async_copy(
                        obuf.at[1 - bank], i_out.at[nbuf.at[1 - bank]],
                        semf).wait()

                return jnp.int32(0), f + 1

            lane0 = iota == 0

            def ent(e, st):
                sl, f = st
                bank = f & 1
                i = clist[pl.ds(e, 16)][0]
                n = cnlist[pl.ds(e, 16)][0]
                lv = jnp.full((16,), i - l0, jnp.int32)
                for k in range(4):
                    cvec = 16 * k + iota
                    u16 = plsc.load_gather(usrc, [cvec, lv])
                    v16 = plsc.load_gather(vsrc, [cvec, lv])
                    obuf[bank, sl, pl.ds(16 * k, 16)] = u16
                    obuf[bank, sl, pl.ds(_D + 16 * k, 16)] = _softplus16(v16)
                plsc.store_scatter(nbuf.at[bank],
                                   [jnp.full((16,), sl, jnp.int32)],
                                   jnp.full((16,), n, jnp.int32), mask=lane0)
                sl = sl + 1
                return lax.cond(sl == 128, flush, lambda s: s, (sl, f))

            return lax.fori_loop(lo, cnt_hi, ent, slot)

        def process_round(woff, total):
            n_entries = jnp.minimum(total - woff, _CAP)
            sort_entries(n_entries)
            slot = (jnp.int32(0), jnp.int32(0))

            def slab_copies(c, ub, vb):
                raw = sel_lo + c * _W
                l0 = pl.multiple_of(jnp.minimum(raw, _LAST_L0), 128)
                csl = pl.ds(l0, _W)
                return [
                    pltpu.make_async_copy(u_t.at[:, csl], ub, semu),
                    pltpu.make_async_copy(v_t.at[:, csl], vb, semv),
                ]

            def ext(c, sl, ub, vb):
                raw = sel_lo + c * _W
                l0 = pl.multiple_of(jnp.minimum(raw, _LAST_L0), 128)
                return extract_entries(c, l0, sl, ub, vb)

            for h in slab_copies(jnp.int32(0), ubuf0, vbuf0):
                h.start()

            def pair(j, sl):
                c0 = 2 * j
                for h in slab_copies(c0 + 1, ubuf1, vbuf1):
                    h.start()
                for h in slab_copies(c0, ubuf0, vbuf0):
                    h.wait()
                sl = ext(c0, sl, ubuf0, vbuf0)

                @pl.when(c0 + 2 < _NCHUNK)
                def _():
                    for h in slab_copies(c0 + 2, ubuf0, vbuf0):
                        h.start()

                for h in slab_copies(c0 + 1, ubuf1, vbuf1):
                    h.wait()
                sl = ext(c0 + 1, sl, ubuf1, vbuf1)
                return sl

            slot = lax.fori_loop(0, _NCHUNK // 2, pair, slot)

            slot = extract_entries(jnp.int32(_NCHUNK), jnp.int32(_TAIL_LO),
                                   slot, utailbuf, vtailbuf)

            sl_end, f_end = slot

            @pl.when(f_end >= 1)
            def _():
                bank = (f_end - 1) & 1
                pltpu.make_async_copy(
                    obuf.at[bank], i_out.at[nbuf.at[bank]], semf).wait()

            @pl.when(sl_end > 0)
            def _():
                lane0 = iota == 0
                bank = f_end & 1
                n0 = nbuf[bank, pl.ds(0, 16)][0]

                def pad(p, carry):
                    plsc.store_scatter(
                        nbuf.at[bank], [jnp.full((16,), p, jnp.int32)],
                        jnp.full((16,), n0, jnp.int32), mask=lane0)
                    for k in range(8):
                        obuf[bank, p, pl.ds(16 * k, 16)] = (
                            obuf[bank, 0, pl.ds(16 * k, 16)])
                    return carry

                lax.fori_loop(sl_end, 128, pad, jnp.int32(0))
                pltpu.sync_copy(obuf.at[bank], i_out.at[nbuf.at[bank]])

        total = scan_select(jnp.int32(0))
        process_round(jnp.int32(0), total)

        def extra_round(r, tot):
            @pl.when(r * _CAP < tot)
            def _():
                t2 = scan_select(r * _CAP)
                process_round(r * _CAP, t2)

            return tot

        lax.fori_loop(1, _N // _CAP, extra_round, total)

    return scan_extract


def _tc_transpose(inter):
    i3 = inter.reshape(2, _B, 2 * _D)

    def body(i_ref, z_ref):
        blk = i_ref[0]
        for s in range(2):
            z_ref[0, s] = blk[:, s * _D:(s + 1) * _D].T

    return pl.pallas_call(
        body,
        grid=(2, _B // 512),
        in_specs=[pl.BlockSpec((1, 512, 2 * _D), lambda i2, b: (i2, b, 0))],
        out_specs=pl.BlockSpec((1, 2, _D, 512), lambda i2, b: (i2, 0, 0, b)),
        out_shape=jax.ShapeDtypeStruct((2, 2, _D, _B), jnp.float32),
    )(i3)


def kernel(idxs, U, V):
    idxf = jnp.transpose(idxs.astype(jnp.int32)).reshape(_N)
    u_t = jnp.transpose(U)
    v_t = jnp.transpose(V)
    u_tail = u_t[:, _TAIL_LO:]
    v_tail = v_t[:, _TAIL_LO:]
    inter = _make_scan_extract()(idxf, u_t, v_t, u_tail, v_tail)
    z = _tc_transpose(inter)
    return jnp.transpose(z, (3, 0, 1, 2))


# speedup vs baseline: 1.1194x; 1.0192x over previous
import functools

import jax
import jax.numpy as jnp
from jax import lax
from jax.experimental import pallas as pl
from jax.experimental.pallas import tpu as pltpu
from jax.experimental.pallas import tpu_sc as plsc

_NL = 1000000
_D = 64
_B = 16384
_N = 2 * _B

_NW = 32
_SEL_W = 31360
_CAP = 2048
_W = 256
_NCHUNK = 124
_TAIL_LO = 999936
_LAST_L0 = _TAIL_LO - _W

_LN2 = 0.6931471805599453
_C3 = 0.3333333432674408
_C5 = 0.2
_C7 = 0.14285714285714285


def _softplus16(x):
    t = jnp.exp(jnp.minimum(x, 20.0))
    z = 1.0 + t
    zi = lax.bitcast_convert_type(z, jnp.int32)
    e = lax.shift_right_arithmetic(zi - 0x3F3504F3, 23)
    m = lax.bitcast_convert_type(zi - lax.shift_left(e, 23), jnp.float32)
    s = (m - 1.0) / (m + 1.0)
    s2 = s * s
    p = 2.0 * s * (1.0 + s2 * (_C3 + s2 * (_C5 + s2 * _C7)))
    ln_z = e.astype(jnp.float32) * _LN2 + p
    return jnp.where(x > 20.0, x, ln_z)


def _iota16():
    return jnp.arange(16, dtype=jnp.int32)


def _make_scan_extract():
    mesh = plsc.VectorSubcoreMesh(core_axis_name="c", subcore_axis_name="s")

    @functools.partial(
        pl.kernel,
        mesh=mesh,
        compiler_params=pltpu.CompilerParams(
            use_tc_tiling_on_sc=True, needs_layout_passes=False),
        out_type=jax.ShapeDtypeStruct((_N, 2 * _D), jnp.float32),
        scratch_types=[
            pltpu.VMEM((4096,), jnp.int32),
            pltpu.VMEM((_CAP + 16,), jnp.int32),
            pltpu.VMEM((_CAP + 16,), jnp.int32),
            pltpu.VMEM((_CAP + 16,), jnp.int32),
            pltpu.VMEM((_CAP + 16,), jnp.int32),
            pltpu.VMEM((256,), jnp.int32),
            pltpu.VMEM((_D, _W), jnp.float32),
            pltpu.VMEM((_D, _W), jnp.float32),
            pltpu.VMEM((_D, _W), jnp.float32),
            pltpu.VMEM((_D, _W), jnp.float32),
            pltpu.VMEM((2, 128, 2 * _D), jnp.float32),
            pltpu.VMEM((2, 128), jnp.int32),
            pltpu.VMEM((_D, _NL - _TAIL_LO), jnp.float32),
            pltpu.VMEM((_D, _NL - _TAIL_LO), jnp.float32),
            pltpu.SemaphoreType.DMA,
            pltpu.SemaphoreType.DMA,
            pltpu.SemaphoreType.DMA,
        ],
    )
    def scan_extract(idxf, u_t, v_t, u_tail, v_tail, i_out, ibuf, ilist,
                     nlist, clist, cnlist, hist, ubuf0, vbuf0, ubuf1, vbuf1,
                     obuf, nbuf, utailbuf, vtailbuf, semu, semv, semf):
        wid = lax.axis_index("s") * 2 + lax.axis_index("c")
        sel_lo = wid * _SEL_W
        sel_hi = jnp.minimum(sel_lo + _SEL_W, _NL)
        iota = _iota16()
        pltpu.sync_copy(u_tail, utailbuf)
        pltpu.sync_copy(v_tail, vtailbuf)

        def scan_select(woff):
            def piece(p, carry):
                off, cbase = carry
                pltpu.sync_copy(idxf.at[pl.ds(p * 4096, 4096)], ibuf)

                def vec(k, carry2):
                    off2, cb2 = carry2
                    v = ibuf[pl.ds(16 * k, 16)]
                    m = (v >= sel_lo) & (v < sel_hi)
                    mi = m.astype(jnp.int32)
                    cnt = plsc.all_reduce_population_count(m)[0]
                    rank = cb2 + plsc.cumsum(mi) - 1
                    m2 = m & (rank >= woff) & (rank < woff + _CAP)
                    nvec = p * 4096 + 16 * k + iota
                    plsc.store_compressed(ilist.at[pl.ds(off2, 16)], v, mask=m2)
                    plsc.store_compressed(nlist.at[pl.ds(off2, 16)], nvec, mask=m2)
                    adv = plsc.all_reduce_population_count(m2)[0]
                    return off2 + adv, cb2 + cnt

                return lax.fori_loop(0, 256, vec, (off, cbase), unroll=2)

            off, total = lax.fori_loop(0, 8, piece, (jnp.int32(0), jnp.int32(0)))
            del off
            return total

        def sort_entries(n_entries):
            zero = jnp.zeros((16,), jnp.int32)
            for k in range(16):
                hist[pl.ds(16 * k, 16)] = zero
            nvecs = (n_entries + 15) // 16

            def hvec(k, carry):
                iv = ilist[pl.ds(16 * k, 16)]
                valid = (16 * k + iota) < n_entries
                cid = jnp.where(iv >= _TAIL_LO, _NCHUNK,
                                lax.shift_right_logical(iv - sel_lo, 8))
                plsc.addupdate_scatter(hist, [cid], jnp.ones((16,), jnp.int32),
                                       mask=valid)
                return carry

            lax.fori_loop(0, nvecs, hvec, jnp.int32(0))

            def pvec(k, carry):
                h = hist[pl.ds(16 * k, 16)]
                c = plsc.cumsum(h)
                excl = carry + c - h
                hist[pl.ds(16 * k, 16)] = excl
                return carry + c[15]

            lax.fori_loop(0, 16, pvec, jnp.int32(0))

            def place(e, carry):
                i = ilist[pl.ds(e, 16)][0]
                n = nlist[pl.ds(e, 16)][0]
                cid = jnp.where(i >= _TAIL_LO, _NCHUNK,
                                lax.shift_right_logical(i - sel_lo, 8))
                pos = hist[pl.ds(cid, 16)][0]
                lane0 = iota == 0
                plsc.store_scatter(clist, [jnp.full((16,), pos, jnp.int32)],
                                   jnp.full((16,), i, jnp.int32), mask=lane0)
                plsc.store_scatter(cnlist, [jnp.full((16,), pos, jnp.int32)],
                                   jnp.full((16,), n, jnp.int32), mask=lane0)
                plsc.store_scatter(hist, [jnp.full((16,), cid, jnp.int32)],
                                   jnp.full((16,), pos + 1, jnp.int32),
                                   mask=lane0)
                return carry

            lax.fori_loop(0, n_entries, place, jnp.int32(0))

        def extract_entries(cid, l0, slot, usrc, vsrc):
            lo = jnp.where(cid > 0, hist[pl.ds(jnp.maximum(cid - 1, 0), 16)][0],
                           0)
            cnt_hi = hist[pl.ds(cid, 16)][0]
            cnt = cnt_hi - lo

            def flush(st):
                sl, f = st
                bank = f & 1
                pltpu.make_async_copy(
                    obuf.at[bank], i_out.at[nbuf.at[bank]], semf).start()

                @pl.when(f >= 1)
                def _():
                    pltpu.make_async_copy(
                        obuf.at[1 - bank], i_out.at[nbuf.at[1 - bank]],
                        semf).wait()

                return jnp.int32(0), f + 1

            lane0 = iota == 0

            def ent(e, st):
                sl, f, i, n = st
                i_nx = clist[pl.ds(e + 1, 16)][0]
                n_nx = cnlist[pl.ds(e + 1, 16)][0]
                bank = f & 1
                lv = jnp.full((16,), i - l0, jnp.int32)
                for k in range(4):
                    cvec = 16 * k + iota
                    u16 = plsc.load_gather(usrc, [cvec, lv])
                    v16 = plsc.load_gather(vsrc, [cvec, lv])
                    obuf[bank, sl, pl.ds(16 * k, 16)] = u16
                    obuf[bank, sl, pl.ds(_D + 16 * k, 16)] = _softplus16(v16)
                plsc.store_scatter(nbuf.at[bank],
                                   [jnp.full((16,), sl, jnp.int32)],
                                   jnp.full((16,), n, jnp.int32), mask=lane0)
                sl = sl + 1
                sl, f = lax.cond(sl == 128, flush, lambda s: s, (sl, f))
                return sl, f, i_nx, n_nx

            sl0, f0 = slot
            i0 = clist[pl.ds(lo, 16)][0]
            n0 = cnlist[pl.ds(lo, 16)][0]
            out = lax.fori_loop(lo, cnt_hi, ent, (sl0, f0, i0, n0))
            return out[0], out[1]

        def process_round(woff, total):
            n_entries = jnp.minimum(total - woff, _CAP)
            sort_entries(n_entries)
            slot = (jnp.int32(0), jnp.int32(0))

            def slab_copies(c, ub, vb):
                raw = sel_lo + c * _W
                l0 = pl.multiple_of(jnp.minimum(raw, _LAST_L0), 128)
                csl = pl.ds(l0, _W)
                return [
                    pltpu.make_async_copy(u_t.at[:, csl], ub, semu),
                    pltpu.make_async_copy(v_t.at[:, csl], vb, semv),
                ]

            def ext(c, sl, ub, vb):
                raw = sel_lo + c * _W
                l0 = pl.multiple_of(jnp.minimum(raw, _LAST_L0), 128)
                return extract_entries(c, l0, sl, ub, vb)

            for h in slab_copies(jnp.int32(0), ubuf0, vbuf0):
                h.start()

            def pair(j, sl):
                c0 = 2 * j
                for h in slab_copies(c0 + 1, ubuf1, vbuf1):
                    h.start()
                for h in slab_copies(c0, ubuf0, vbuf0):
                    h.wait()
                sl = ext(c0, sl, ubuf0, vbuf0)

                @pl.when(c0 + 2 < _NCHUNK)
                def _():
                    for h in slab_copies(c0 + 2, ubuf0, vbuf0):
                        h.start()

                for h in slab_copies(c0 + 1, ubuf1, vbuf1):
                    h.wait()
                sl = ext(c0 + 1, sl, ubuf1, vbuf1)
                return sl

            slot = lax.fori_loop(0, _NCHUNK // 2, pair, slot)

            slot = extract_entries(jnp.int32(_NCHUNK), jnp.int32(_TAIL_LO),
                                   slot, utailbuf, vtailbuf)

            sl_end, f_end = slot

            @pl.when(f_end >= 1)
            def _():
                bank = (f_end - 1) & 1
                pltpu.make_async_copy(
                    obuf.at[bank], i_out.at[nbuf.at[bank]], semf).wait()

            @pl.when(sl_end > 0)
            def _():
                lane0 = iota == 0
                bank = f_end & 1
                n0 = nbuf[bank, pl.ds(0, 16)][0]

                def pad(p, carry):
                    plsc.store_scatter(
                        nbuf.at[bank], [jnp.full((16,), p, jnp.int32)],
                        jnp.full((16,), n0, jnp.int32), mask=lane0)
                    for k in range(8):
                        obuf[bank, p, pl.ds(16 * k, 16)] = (
                            obuf[bank, 0, pl.ds(16 * k, 16)])
                    return carry

                lax.fori_loop(sl_end, 128, pad, jnp.int32(0))
                pltpu.sync_copy(obuf.at[bank], i_out.at[nbuf.at[bank]])

        total = scan_select(jnp.int32(0))
        process_round(jnp.int32(0), total)

        def extra_round(r, tot):
            @pl.when(r * _CAP < tot)
            def _():
                t2 = scan_select(r * _CAP)
                process_round(r * _CAP, t2)

            return tot

        lax.fori_loop(1, _N // _CAP, extra_round, total)

    return scan_extract


def _tc_transpose(inter):
    i3 = inter.reshape(2, _B, 2 * _D)

    def body(i_ref, z_ref):
        blk = i_ref[0]
        for s in range(2):
            z_ref[0, s] = blk[:, s * _D:(s + 1) * _D].T

    return pl.pallas_call(
        body,
        grid=(2, _B // 512),
        in_specs=[pl.BlockSpec((1, 512, 2 * _D), lambda i2, b: (i2, b, 0))],
        out_specs=pl.BlockSpec((1, 2, _D, 512), lambda i2, b: (i2, 0, 0, b)),
        out_shape=jax.ShapeDtypeStruct((2, 2, _D, _B), jnp.float32),
    )(i3)


def kernel(idxs, U, V):
    idxf = jnp.transpose(idxs.astype(jnp.int32)).reshape(_N)
    u_t = jnp.transpose(U)
    v_t = jnp.transpose(V)
    u_tail = u_t[:, _TAIL_LO:]
    v_tail = v_t[:, _TAIL_LO:]
    inter = _make_scan_extract()(idxf, u_t, v_t, u_tail, v_tail)
    z = _tc_transpose(inter)
    return jnp.transpose(z, (3, 0, 1, 2))


# speedup vs baseline: 1.1590x; 1.0354x over previous
import functools

import jax
import jax.numpy as jnp
from jax import lax
from jax.experimental import pallas as pl
from jax.experimental.pallas import tpu as pltpu
from jax.experimental.pallas import tpu_sc as plsc

_NL = 1000000
_D = 64
_B = 16384
_N = 2 * _B

_NW = 32
_SEL_W = 31360
_CAP = 2048
_W = 256
_NCHUNK = 124
_TAIL_LO = 999936
_LAST_L0 = _TAIL_LO - _W

_LN2 = 0.6931471805599453
_C3 = 0.3333333432674408
_C5 = 0.2
_C7 = 0.14285714285714285


def _softplus16(x):
    t = jnp.exp(jnp.minimum(x, 20.0))
    z = 1.0 + t
    zi = lax.bitcast_convert_type(z, jnp.int32)
    e = lax.shift_right_arithmetic(zi - 0x3F3504F3, 23)
    m = lax.bitcast_convert_type(zi - lax.shift_left(e, 23), jnp.float32)
    s = (m - 1.0) / (m + 1.0)
    s2 = s * s
    p = 2.0 * s * (1.0 + s2 * (_C3 + s2 * (_C5 + s2 * _C7)))
    ln_z = e.astype(jnp.float32) * _LN2 + p
    return jnp.where(x > 20.0, x, ln_z)


def _iota16():
    return jnp.arange(16, dtype=jnp.int32)


def _make_scan_extract():
    mesh = plsc.VectorSubcoreMesh(core_axis_name="c", subcore_axis_name="s")

    @functools.partial(
        pl.kernel,
        mesh=mesh,
        compiler_params=pltpu.CompilerParams(
            use_tc_tiling_on_sc=True, needs_layout_passes=False),
        out_type=jax.ShapeDtypeStruct((_N, 2 * _D), jnp.float32),
        scratch_types=[
            pltpu.VMEM((4096,), jnp.int32),
            pltpu.VMEM((_CAP + 16,), jnp.int32),
            pltpu.VMEM((_CAP + 16,), jnp.int32),
            pltpu.VMEM((_CAP + 16,), jnp.int32),
            pltpu.VMEM((_CAP + 16,), jnp.int32),
            pltpu.VMEM((256,), jnp.int32),
            pltpu.VMEM((_D, _W), jnp.float32),
            pltpu.VMEM((_D, _W), jnp.float32),
            pltpu.VMEM((_D, _W), jnp.float32),
            pltpu.VMEM((_D, _W), jnp.float32),
            pltpu.VMEM((2, 128, 2 * _D), jnp.float32),
            pltpu.VMEM((2, 128), jnp.int32),
            pltpu.VMEM((_D, _NL - _TAIL_LO), jnp.float32),
            pltpu.VMEM((_D, _NL - _TAIL_LO), jnp.float32),
            pltpu.SemaphoreType.DMA,
            pltpu.SemaphoreType.DMA,
            pltpu.SemaphoreType.DMA,
        ],
    )
    def scan_extract(idxf, u_t, v_t, u_tail, v_tail, i_out, ibuf, ilist,
                     nlist, clist, cnlist, hist, ubuf0, vbuf0, ubuf1, vbuf1,
                     obuf, nbuf, utailbuf, vtailbuf, semu, semv, semf):
        wid = lax.axis_index("s") * 2 + lax.axis_index("c")
        sel_lo = wid * _SEL_W
        sel_hi = jnp.minimum(sel_lo + _SEL_W, _NL)
        iota = _iota16()
        pltpu.sync_copy(u_tail, utailbuf)
        pltpu.sync_copy(v_tail, vtailbuf)

        def scan_select(woff):
            def piece(p, carry):
                off, cbase = carry
                pltpu.sync_copy(idxf.at[pl.ds(p * 4096, 4096)], ibuf)

                def vec(k, carry2):
                    off2, cb2 = carry2
                    v = ibuf[pl.ds(16 * k, 16)]
                    m = (v >= sel_lo) & (v < sel_hi)
                    mi = m.astype(jnp.int32)
                    cnt = plsc.all_reduce_population_count(m)[0]
                    rank = cb2 + plsc.cumsum(mi) - 1
                    m2 = m & (rank >= woff) & (rank < woff + _CAP)
                    nvec = p * 4096 + 16 * k + iota
                    plsc.store_compressed(ilist.at[pl.ds(off2, 16)], v, mask=m2)
                    plsc.store_compressed(nlist.at[pl.ds(off2, 16)], nvec, mask=m2)
                    adv = plsc.all_reduce_population_count(m2)[0]
                    return off2 + adv, cb2 + cnt

                return lax.fori_loop(0, 256, vec, (off, cbase), unroll=2)

            off, total = lax.fori_loop(0, 8, piece, (jnp.int32(0), jnp.int32(0)))
            del off
            return total

        def sort_entries(n_entries):
            zero = jnp.zeros((16,), jnp.int32)
            for k in range(16):
                hist[pl.ds(16 * k, 16)] = zero
            nvecs = (n_entries + 15) // 16

            def hvec(k, carry):
                iv = ilist[pl.ds(16 * k, 16)]
                valid = (16 * k + iota) < n_entries
                cid = jnp.where(iv >= _TAIL_LO, _NCHUNK,
                                lax.shift_right_logical(iv - sel_lo, 8))
                plsc.addupdate_scatter(hist, [cid], jnp.ones((16,), jnp.int32),
                                       mask=valid)
                return carry

            lax.fori_loop(0, nvecs, hvec, jnp.int32(0))

            def pvec(k, carry):
                h = hist[pl.ds(16 * k, 16)]
                c = plsc.cumsum(h)
                excl = carry + c - h
                hist[pl.ds(16 * k, 16)] = excl
                return carry + c[15]

            lax.fori_loop(0, 16, pvec, jnp.int32(0))

            lane0 = iota == 0

            def place(e, carry):
                i, n = carry
                i_nx = ilist[pl.ds(e + 1, 16)][0]
                n_nx = nlist[pl.ds(e + 1, 16)][0]
                cid = jnp.where(i >= _TAIL_LO, _NCHUNK,
                                lax.shift_right_logical(i - sel_lo, 8))
                pos = hist[pl.ds(cid, 16)][0]
                plsc.store_scatter(clist, [jnp.full((16,), pos, jnp.int32)],
                                   jnp.full((16,), i, jnp.int32), mask=lane0)
                plsc.store_scatter(cnlist, [jnp.full((16,), pos, jnp.int32)],
                                   jnp.full((16,), n, jnp.int32), mask=lane0)
                plsc.store_scatter(hist, [jnp.full((16,), cid, jnp.int32)],
                                   jnp.full((16,), pos + 1, jnp.int32),
                                   mask=lane0)
                return i_nx, n_nx

            lax.fori_loop(0, n_entries, place,
                          (ilist[pl.ds(0, 16)][0], nlist[pl.ds(0, 16)][0]))

        def extract_entries(cid, l0, slot, usrc, vsrc):
            lo = jnp.where(cid > 0, hist[pl.ds(jnp.maximum(cid - 1, 0), 16)][0],
                           0)
            cnt_hi = hist[pl.ds(cid, 16)][0]
            cnt = cnt_hi - lo

            def flush(st):
                sl, f = st
                bank = f & 1
                pltpu.make_async_copy(
                    obuf.at[bank], i_out.at[nbuf.at[bank]], semf).start()

                @pl.when(f >= 1)
                def _():
                    pltpu.make_async_copy(
                        obuf.at[1 - bank], i_out.at[nbuf.at[1 - bank]],
                        semf).wait()

                return jnp.int32(0), f + 1

            lane0 = iota == 0

            def ent(e, st):
                sl, f, i, n = st
                i_nx = clist[pl.ds(e + 1, 16)][0]
                n_nx = cnlist[pl.ds(e + 1, 16)][0]
                bank = f & 1
                lv = jnp.full((16,), i - l0, jnp.int32)
                for k in range(4):
                    cvec = 16 * k + iota
                    u16 = plsc.load_gather(usrc, [cvec, lv])
                    v16 = plsc.load_gather(vsrc, [cvec, lv])
                    obuf[bank, sl, pl.ds(16 * k, 16)] = u16
                    obuf[bank, sl, pl.ds(_D + 16 * k, 16)] = _softplus16(v16)
                plsc.store_scatter(nbuf.at[bank],
                                   [jnp.full((16,), sl, jnp.int32)],
                                   jnp.full((16,), n, jnp.int32), mask=lane0)
                sl = sl + 1
                sl, f = lax.cond(sl == 128, flush, lambda s: s, (sl, f))
                return sl, f, i_nx, n_nx

            sl0, f0 = slot
            i0 = clist[pl.ds(lo, 16)][0]
            n0 = cnlist[pl.ds(lo, 16)][0]
            out = lax.fori_loop(lo, cnt_hi, ent, (sl0, f0, i0, n0))
            return out[0], out[1]

        def process_round(woff, total):
            n_entries = jnp.minimum(total - woff, _CAP)
            sort_entries(n_entries)
            slot = (jnp.int32(0), jnp.int32(0))

            def slab_copies(c, ub, vb):
                raw = sel_lo + c * _W
                l0 = pl.multiple_of(jnp.minimum(raw, _LAST_L0), 128)
                csl = pl.ds(l0, _W)
                return [
                    pltpu.make_async_copy(u_t.at[:, csl], ub, semu),
                    pltpu.make_async_copy(v_t.at[:, csl], vb, semv),
                ]

            def ext(c, sl, ub, vb):
                raw = sel_lo + c * _W
                l0 = pl.multiple_of(jnp.minimum(raw, _LAST_L0), 128)
                return extract_entries(c, l0, sl, ub, vb)

            for h in slab_copies(jnp.int32(0), ubuf0, vbuf0):
                h.start()

            def pair(j, sl):
                c0 = 2 * j
                for h in slab_copies(c0 + 1, ubuf1, vbuf1):
                    h.start()
                for h in slab_copies(c0, ubuf0, vbuf0):
                    h.wait()
                sl = ext(c0, sl, ubuf0, vbuf0)

                @pl.when(c0 + 2 < _NCHUNK)
                def _():
                    for h in slab_copies(c0 + 2, ubuf0, vbuf0):
                        h.start()

                for h in slab_copies(c0 + 1, ubuf1, vbuf1):
                    h.wait()
                sl = ext(c0 + 1, sl, ubuf1, vbuf1)
                return sl

            slot = lax.fori_loop(0, _NCHUNK // 2, pair, slot)

            slot = extract_entries(jnp.int32(_NCHUNK), jnp.int32(_TAIL_LO),
                                   slot, utailbuf, vtailbuf)

            sl_end, f_end = slot

            @pl.when(f_end >= 1)
            def _():
                bank = (f_end - 1) & 1
                pltpu.make_async_copy(
                    obuf.at[bank], i_out.at[nbuf.at[bank]], semf).wait()

            @pl.when(sl_end > 0)
            def _():
                lane0 = iota == 0
                bank = f_end & 1
                n0 = nbuf[bank, pl.ds(0, 16)][0]

                def pad(p, carry):
                    plsc.store_scatter(
                        nbuf.at[bank], [jnp.full((16,), p, jnp.int32)],
                        jnp.full((16,), n0, jnp.int32), mask=lane0)
                    for k in range(8):
                        obuf[bank, p, pl.ds(16 * k, 16)] = (
                            obuf[bank, 0, pl.ds(16 * k, 16)])
                    return carry

                lax.fori_loop(sl_end, 128, pad, jnp.int32(0))
                pltpu.sync_copy(obuf.at[bank], i_out.at[nbuf.at[bank]])

        total = scan_select(jnp.int32(0))
        process_round(jnp.int32(0), total)

        def extra_round(r, tot):
            @pl.when(r * _CAP < tot)
            def _():
                t2 = scan_select(r * _CAP)
                process_round(r * _CAP, t2)

            return tot

        lax.fori_loop(1, _N // _CAP, extra_round, total)

    return scan_extract


def _tc_transpose(inter):
    i3 = inter.reshape(2, _B, 2 * _D)

    def body(i_ref, z_ref):
        blk = i_ref[0]
        for s in range(2):
            z_ref[0, s] = blk[:, s * _D:(s + 1) * _D].T

    return pl.pallas_call(
        body,
        grid=(2, _B // 512),
        in_specs=[pl.BlockSpec((1, 512, 2 * _D), lambda i2, b: (i2, b, 0))],
        out_specs=pl.BlockSpec((1, 2, _D, 512), lambda i2, b: (i2, 0, 0, b)),
        out_shape=jax.ShapeDtypeStruct((2, 2, _D, _B), jnp.float32),
    )(i3)


def kernel(idxs, U, V):
    idxf = jnp.transpose(idxs.astype(jnp.int32)).reshape(_N)
    u_t = jnp.transpose(U)
    v_t = jnp.transpose(V)
    u_tail = u_t[:, _TAIL_LO:]
    v_tail = v_t[:, _TAIL_LO:]
    inter = _make_scan_extract()(idxf, u_t, v_t, u_tail, v_tail)
    z = _tc_transpose(inter)
    return jnp.transpose(z, (3, 0, 1, 2))


# speedup vs baseline: 1.1729x; 1.0120x over previous
import functools

import jax
import jax.numpy as jnp
from jax import lax
from jax.experimental import pallas as pl
from jax.experimental.pallas import tpu as pltpu
from jax.experimental.pallas import tpu_sc as plsc

_NL = 1000000
_D = 64
_B = 16384
_N = 2 * _B

_NW = 32
_SEL_W = 31360
_CAP = 2048
_W = 256
_NCHUNK = 124
_TAIL_LO = 999936
_LAST_L0 = _TAIL_LO - _W

_LN2 = 0.6931471805599453
_C3 = 0.3333333432674408
_C5 = 0.2
_C7 = 0.14285714285714285


def _softplus16(x):
    t = jnp.exp(jnp.minimum(x, 20.0))
    z = 1.0 + t
    zi = lax.bitcast_convert_type(z, jnp.int32)
    e = lax.shift_right_arithmetic(zi - 0x3F3504F3, 23)
    m = lax.bitcast_convert_type(zi - lax.shift_left(e, 23), jnp.float32)
    s = (m - 1.0) / (m + 1.0)
    s2 = s * s
    p = 2.0 * s * (1.0 + s2 * (_C3 + s2 * (_C5 + s2 * _C7)))
    ln_z = e.astype(jnp.float32) * _LN2 + p
    return jnp.where(x > 20.0, x, ln_z)


def _iota16():
    return jnp.arange(16, dtype=jnp.int32)


def _make_scan_extract():
    mesh = plsc.VectorSubcoreMesh(core_axis_name="c", subcore_axis_name="s")

    @functools.partial(
        pl.kernel,
        mesh=mesh,
        compiler_params=pltpu.CompilerParams(
            use_tc_tiling_on_sc=True, needs_layout_passes=False),
        out_type=jax.ShapeDtypeStruct((_N, 2 * _D), jnp.float32),
        scratch_types=[
            pltpu.VMEM((2, 2048), jnp.int32),
            pltpu.VMEM((_CAP + 16,), jnp.int32),
            pltpu.VMEM((_CAP + 16,), jnp.int32),
            pltpu.VMEM((_CAP + 16,), jnp.int32),
            pltpu.VMEM((_CAP + 16,), jnp.int32),
            pltpu.VMEM((256,), jnp.int32),
            pltpu.VMEM((_D, _W), jnp.float32),
            pltpu.VMEM((_D, _W), jnp.float32),
            pltpu.VMEM((_D, _W), jnp.float32),
            pltpu.VMEM((_D, _W), jnp.float32),
            pltpu.VMEM((2, 128, 2 * _D), jnp.float32),
            pltpu.VMEM((2, 128), jnp.int32),
            pltpu.VMEM((_D, _NL - _TAIL_LO), jnp.float32),
            pltpu.VMEM((_D, _NL - _TAIL_LO), jnp.float32),
            pltpu.SemaphoreType.DMA,
            pltpu.SemaphoreType.DMA,
            pltpu.SemaphoreType.DMA,
        ],
    )
    def scan_extract(idxf, u_t, v_t, u_tail, v_tail, i_out, ibuf, ilist,
                     nlist, clist, cnlist, hist, ubuf0, vbuf0, ubuf1, vbuf1,
                     obuf, nbuf, utailbuf, vtailbuf, semu, semv, semf):
        wid = lax.axis_index("s") * 2 + lax.axis_index("c")
        sel_lo = wid * _SEL_W
        sel_hi = jnp.minimum(sel_lo + _SEL_W, _NL)
        iota = _iota16()
        pltpu.sync_copy(u_tail, utailbuf)
        pltpu.sync_copy(v_tail, vtailbuf)

        def scan_select(woff):
            def stage(p, bank):
                return pltpu.make_async_copy(
                    idxf.at[pl.ds(p * 2048, 2048)], ibuf.at[bank], semf)

            stage(jnp.int32(0), jnp.int32(0)).start()

            def piece(p, carry):
                off, cbase = carry
                bank = p & 1
                stage(p, bank).wait()

                @pl.when(p + 1 < 16)
                def _():
                    stage(p + 1, 1 - bank).start()

                def vec(k, carry2):
                    off2, cb2 = carry2
                    v = ibuf[bank, pl.ds(16 * k, 16)]
                    m = (v >= sel_lo) & (v < sel_hi)
                    mi = m.astype(jnp.int32)
                    cnt = plsc.all_reduce_population_count(m)[0]
                    rank = cb2 + plsc.cumsum(mi) - 1
                    m2 = m & (rank >= woff) & (rank < woff + _CAP)
                    nvec = p * 2048 + 16 * k + iota
                    plsc.store_compressed(ilist.at[pl.ds(off2, 16)], v, mask=m2)
                    plsc.store_compressed(nlist.at[pl.ds(off2, 16)], nvec, mask=m2)
                    adv = plsc.all_reduce_population_count(m2)[0]
                    return off2 + adv, cb2 + cnt

                return lax.fori_loop(0, 128, vec, (off, cbase), unroll=2)

            off, total = lax.fori_loop(0, 16, piece, (jnp.int32(0), jnp.int32(0)))
            del off
            return total

        def sort_entries(n_entries):
            zero = jnp.zeros((16,), jnp.int32)
            for k in range(16):
                hist[pl.ds(16 * k, 16)] = zero
            nvecs = (n_entries + 15) // 16

            def hvec(k, carry):
                iv = ilist[pl.ds(16 * k, 16)]
                valid = (16 * k + iota) < n_entries
                cid = jnp.where(iv >= _TAIL_LO, _NCHUNK,
                                lax.shift_right_logical(iv - sel_lo, 8))
                plsc.addupdate_scatter(hist, [cid], jnp.ones((16,), jnp.int32),
                                       mask=valid)
                return carry

            lax.fori_loop(0, nvecs, hvec, jnp.int32(0))

            def pvec(k, carry):
                h = hist[pl.ds(16 * k, 16)]
                c = plsc.cumsum(h)
                excl = carry + c - h
                hist[pl.ds(16 * k, 16)] = excl
                return carry + c[15]

            lax.fori_loop(0, 16, pvec, jnp.int32(0))

            lane0 = iota == 0

            def place(e, carry):
                i, n = carry
                i_nx = ilist[pl.ds(e + 1, 16)][0]
                n_nx = nlist[pl.ds(e + 1, 16)][0]
                cid = jnp.where(i >= _TAIL_LO, _NCHUNK,
                                lax.shift_right_logical(i - sel_lo, 8))
                pos = hist[pl.ds(cid, 16)][0]
                plsc.store_scatter(clist, [jnp.full((16,), pos, jnp.int32)],
                                   jnp.full((16,), i, jnp.int32), mask=lane0)
                plsc.store_scatter(cnlist, [jnp.full((16,), pos, jnp.int32)],
                                   jnp.full((16,), n, jnp.int32), mask=lane0)
                plsc.store_scatter(hist, [jnp.full((16,), cid, jnp.int32)],
                                   jnp.full((16,), pos + 1, jnp.int32),
                                   mask=lane0)
                return i_nx, n_nx

            lax.fori_loop(0, n_entries, place,
                          (ilist[pl.ds(0, 16)][0], nlist[pl.ds(0, 16)][0]))

        def extract_entries(cid, l0, slot, usrc, vsrc):
            lo = jnp.where(cid > 0, hist[pl.ds(jnp.maximum(cid - 1, 0), 16)][0],
                           0)
            cnt_hi = hist[pl.ds(cid, 16)][0]
            cnt = cnt_hi - lo

            def flush(st):
                sl, f = st
                bank = f & 1
                pltpu.make_async_copy(
                    obuf.at[bank], i_out.at[nbuf.at[bank]], semf).start()

                @pl.when(f >= 1)
                def _():
                    pltpu.make_async_copy(
                        obuf.at[1 - bank], i_out.at[nbuf.at[1 - bank]],
                        semf).wait()

                return jnp.int32(0), f + 1

            lane0 = iota == 0

            def ent(e, st):
                sl, f, i, n = st
                i_nx = clist[pl.ds(e + 1, 16)][0]
                n_nx = cnlist[pl.ds(e + 1, 16)][0]
                bank = f & 1
                lv = jnp.full((16,), i - l0, jnp.int32)
                for k in range(4):
                    cvec = 16 * k + iota
                    u16 = plsc.load_gather(usrc, [cvec, lv])
                    v16 = plsc.load_gather(vsrc, [cvec, lv])
                    obuf[bank, sl, pl.ds(16 * k, 16)] = u16
                    obuf[bank, sl, pl.ds(_D + 16 * k, 16)] = _softplus16(v16)
                plsc.store_scatter(nbuf.at[bank],
                                   [jnp.full((16,), sl, jnp.int32)],
                                   jnp.full((16,), n, jnp.int32), mask=lane0)
                sl = sl + 1
                sl, f = lax.cond(sl == 128, flush, lambda s: s, (sl, f))
                return sl, f, i_nx, n_nx

            sl0, f0 = slot
            i0 = clist[pl.ds(lo, 16)][0]
            n0 = cnlist[pl.ds(lo, 16)][0]
            out = lax.fori_loop(lo, cnt_hi, ent, (sl0, f0, i0, n0))
            return out[0], out[1]

        def process_round(woff, total):
            n_entries = jnp.minimum(total - woff, _CAP)
            sort_entries(n_entries)
            slot = (jnp.int32(0), jnp.int32(0))

            def slab_copies(c, ub, vb):
                raw = sel_lo + c * _W
                l0 = pl.multiple_of(jnp.minimum(raw, _LAST_L0), 128)
                csl = pl.ds(l0, _W)
                return [
                    pltpu.make_async_copy(u_t.at[:, csl], ub, semu),
                    pltpu.make_async_copy(v_t.at[:, csl], vb, semv),
                ]

            def ext(c, sl, ub, vb):
                raw = sel_lo + c * _W
                l0 = pl.multiple_of(jnp.minimum(raw, _LAST_L0), 128)
                return extract_entries(c, l0, sl, ub, vb)

            for h in slab_copies(jnp.int32(0), ubuf0, vbuf0):
                h.start()

            def pair(j, sl):
                c0 = 2 * j
                for h in slab_copies(c0 + 1, ubuf1, vbuf1):
                    h.start()
                for h in slab_copies(c0, ubuf0, vbuf0):
                    h.wait()
                sl = ext(c0, sl, ubuf0, vbuf0)

                @pl.when(c0 + 2 < _NCHUNK)
                def _():
                    for h in slab_copies(c0 + 2, ubuf0, vbuf0):
                        h.start()

                for h in slab_copies(c0 + 1, ubuf1, vbuf1):
                    h.wait()
                sl = ext(c0 + 1, sl, ubuf1, vbuf1)
                return sl

            slot = lax.fori_loop(0, _NCHUNK // 2, pair, slot)

            slot = extract_entries(jnp.int32(_NCHUNK), jnp.int32(_TAIL_LO),
                                   slot, utailbuf, vtailbuf)

            sl_end, f_end = slot

            @pl.when(f_end >= 1)
            def _():
                bank = (f_end - 1) & 1
                pltpu.make_async_copy(
                    obuf.at[bank], i_out.at[nbuf.at[bank]], semf).wait()

            @pl.when(sl_end > 0)
            def _():
                lane0 = iota == 0
                bank = f_end & 1
                n0 = nbuf[bank, pl.ds(0, 16)][0]

                def pad(p, carry):
                    plsc.store_scatter(
                        nbuf.at[bank], [jnp.full((16,), p, jnp.int32)],
                        jnp.full((16,), n0, jnp.int32), mask=lane0)
                    for k in range(8):
                        obuf[bank, p, pl.ds(16 * k, 16)] = (
                            obuf[bank, 0, pl.ds(16 * k, 16)])
                    return carry

                lax.fori_loop(sl_end, 128, pad, jnp.int32(0))
                pltpu.sync_copy(obuf.at[bank], i_out.at[nbuf.at[bank]])

        total = scan_select(jnp.int32(0))
        process_round(jnp.int32(0), total)

        def extra_round(r, tot):
            @pl.when(r * _CAP < tot)
            def _():
                t2 = scan_select(r * _CAP)
                process_round(r * _CAP, t2)

            return tot

        lax.fori_loop(1, _N // _CAP, extra_round, total)

    return scan_extract


def _tc_transpose(inter):
    i3 = inter.reshape(2, _B, 2 * _D)

    def body(i_ref, z_ref):
        blk = i_ref[0]
        for s in range(2):
            z_ref[0, s] = blk[:, s * _D:(s + 1) * _D].T

    return pl.pallas_call(
        body,
        grid=(2, _B // 512),
        in_specs=[pl.BlockSpec((1, 512, 2 * _D), lambda i2, b: (i2, b, 0))],
        out_specs=pl.BlockSpec((1, 2, _D, 512), lambda i2, b: (i2, 0, 0, b)),
        out_shape=jax.ShapeDtypeStruct((2, 2, _D, _B), jnp.float32),
    )(i3)


def kernel(idxs, U, V):
    idxf = jnp.transpose(idxs.astype(jnp.int32)).reshape(_N)
    u_t = jnp.transpose(U)
    v_t = jnp.transpose(V)
    u_tail = u_t[:, _TAIL_LO:]
    v_tail = v_t[:, _TAIL_LO:]
    inter = _make_scan_extract()(idxf, u_t, v_t, u_tail, v_tail)
    z = _tc_transpose(inter)
    return jnp.transpose(z, (3, 0, 1, 2))
